# Initial kernel scaffold; baseline (speedup 1.0000x reference)
import jax
import jax.numpy as jnp
from jax.experimental import pallas as pl


def kernel(x, edge_index, edge_attr, params):
    raise NotImplementedError("write your pallas kernel here")


# speedup vs baseline: 13.0605x; 13.0605x over previous
import functools

import jax
import jax.numpy as jnp
from jax import lax
from jax.experimental import pallas as pl
from jax.experimental.pallas import tpu as pltpu
from jax.experimental.pallas import tpu_sc as plsc

N_NODES = 100000
N_PAD = 100096
NQ = N_PAD // 16
E = 6400000
EQ = E // 16
ER = E // 128
NW = 32
RPW = ER // NW
KC = 22
NCH = RPW // KC
CHE = KC * 128
TAIL0 = NW * RPW
NSEG = N_PAD // 16


@functools.lru_cache(maxsize=None)
def _mesh():
    return plsc.VectorSubcoreMesh(core_axis_name="c", subcore_axis_name="s")


def _wid():
    return lax.axis_index("s") * 2 + lax.axis_index("c")


def _gather_chunk(tbl, idx2d, out_hbm, idx_v, rows_v, sem, row0, nrows):
    pltpu.sync_copy(idx2d.at[pl.ds(row0, nrows), :],
                    idx_v.at[pl.ds(0, nrows), :])
    cps = []
    for j in range(nrows):
        cps.append(pltpu.async_copy(
            tbl.at[idx_v.at[j]],
            rows_v.at[pl.ds(j * 128, 128), :], sem))
    for cp in cps:
        cp.wait()
    pltpu.sync_copy(rows_v.at[pl.ds(0, nrows * 128), :],
                    out_hbm.at[pl.ds(row0 * 128, nrows * 128), :])


def _gather_body(x_tbl, row2d, col2d, xr_out, xc_out,
                 idx_v, rows_v, idx2_v, rows2_v, sem1, sem2):
    w = _wid()
    base = w * RPW

    def step(i, carry):
        r0 = base + i * KC
        _gather_chunk(x_tbl, row2d, xr_out, idx_v, rows_v, sem1, r0, KC)
        _gather_chunk(x_tbl, col2d, xc_out, idx2_v, rows2_v, sem2, r0, KC)
        return carry

    lax.fori_loop(0, NCH, step, 0)

    @pl.when(w < ER - TAIL0)
    def _tail():
        r0 = TAIL0 + w
        _gather_chunk(x_tbl, row2d, xr_out, idx_v, rows_v, sem1, r0, 1)
        _gather_chunk(x_tbl, col2d, xc_out, idx2_v, rows2_v, sem2, r0, 1)


@functools.lru_cache(maxsize=None)
def _gather_kernel():
    return pl.kernel(
        _gather_body,
        out_type=[jax.ShapeDtypeStruct((E, 8), jnp.float32),
                  jax.ShapeDtypeStruct((E, 8), jnp.float32)],
        mesh=_mesh(),
        compiler_params=pltpu.CompilerParams(use_tc_tiling_on_sc=False),
        scratch_types=[
            pltpu.VMEM((KC, 128), jnp.int32),
            pltpu.VMEM((CHE, 8), jnp.float32),
            pltpu.VMEM((KC, 128), jnp.int32),
            pltpu.VMEM((CHE, 8), jnp.float32),
            pltpu.SemaphoreType.DMA,
            pltpu.SemaphoreType.DMA,
        ],
    )


def _scatter_chunk(m_hbm, idx2d, acc_s, cnt_s, idx_v, upd_v, ones_v,
                   row0, nrows, with_counts):
    pltpu.sync_copy(idx2d.at[pl.ds(row0, nrows), :],
                    idx_v.at[pl.ds(0, nrows), :])
    pltpu.sync_copy(m_hbm.at[pl.ds(row0 * 128, nrows * 128), :],
                    upd_v.at[pl.ds(0, nrows * 128), :])
    for j in range(nrows):
        pltpu.sync_copy(upd_v.at[pl.ds(j * 128, 128), :],
                        acc_s.at[idx_v.at[j]], add=True)
        if with_counts:
            pltpu.sync_copy(ones_v, cnt_s.at[idx_v.at[j]], add=True)


def _scatter_body(m_hbm, col2d, zeros_hbm, ones_hbm, out_hbm, cnt_hbm,
                  acc_s, cnt_s, idx_v, upd_v, ones_v, with_counts):
    c = lax.axis_index("c")
    s = lax.axis_index("s")
    w = _wid()
    pltpu.sync_copy(zeros_hbm.at[pl.ds(s * NSEG, NSEG), :],
                    acc_s.at[pl.ds(s * NSEG, NSEG), :])
    if with_counts:
        pltpu.sync_copy(zeros_hbm.at[pl.ds(s * NSEG, NSEG), :],
                        cnt_s.at[pl.ds(s * NSEG, NSEG), :])
        pltpu.sync_copy(ones_hbm, ones_v)
    plsc.subcore_barrier()
    base = w * RPW

    def step(i, carry):
        _scatter_chunk(m_hbm, col2d, acc_s, cnt_s, idx_v, upd_v, ones_v,
                       base + i * KC, KC, with_counts)
        return carry

    lax.fori_loop(0, NCH, step, 0)

    @pl.when(w < ER - TAIL0)
    def _tail():
        _scatter_chunk(m_hbm, col2d, acc_s, cnt_s, idx_v, upd_v, ones_v,
                       TAIL0 + w, 1, with_counts)

    plsc.subcore_barrier()
    pltpu.sync_copy(acc_s.at[pl.ds(s * NSEG, NSEG), :],
                    out_hbm.at[c, pl.ds(s * NSEG, NSEG), :])
    if with_counts:
        pltpu.sync_copy(cnt_s.at[pl.ds(s * NSEG, NSEG), :],
                        cnt_hbm.at[c, pl.ds(s * NSEG, NSEG), :])


@functools.lru_cache(maxsize=None)
def _scatter_kernel(with_counts):
    n_out = 2 if with_counts else 1
    if with_counts:
        def body(m_hbm, col2d, zeros_hbm, ones_hbm, out_hbm, cnt_hbm,
                 acc_s, cnt_s, idx_v, upd_v, ones_v):
            return _scatter_body(m_hbm, col2d, zeros_hbm, ones_hbm,
                                 out_hbm, cnt_hbm,
                                 acc_s, cnt_s, idx_v, upd_v, ones_v, True)
    else:
        def body(m_hbm, col2d, zeros_hbm, out_hbm,
                 acc_s, idx_v, upd_v):
            return _scatter_body(m_hbm, col2d, zeros_hbm, None,
                                 out_hbm, None,
                                 acc_s, None, idx_v, upd_v, None, False)
    out_type = [jax.ShapeDtypeStruct((2, N_PAD, 8), jnp.float32)] * n_out
    scratch = [pltpu.VMEM_SHARED((N_PAD, 8), jnp.float32)]
    if with_counts:
        scratch.append(pltpu.VMEM_SHARED((N_PAD, 8), jnp.float32))
    scratch += [
        pltpu.VMEM((KC, 128), jnp.int32),
        pltpu.VMEM((CHE, 8), jnp.float32),
    ]
    if with_counts:
        scratch.append(pltpu.VMEM((128, 8), jnp.float32))
    return pl.kernel(body, out_type=out_type, mesh=_mesh(),
                     compiler_params=pltpu.CompilerParams(
                         use_tc_tiling_on_sc=False),
                     scratch_types=scratch)


def _edge_mlp_body(xr, xc, ea, g1r, g1c, g1e, b1, g2en, b2en,
                   gnr, gne, bn1, gm2, bm2, zout):
    zr = xr[...]
    zc = xc[...]
    ze = ea[...]
    dot = functools.partial(jnp.dot, preferred_element_type=jnp.float32)
    h = jnp.maximum(dot(zr, g1r[...]) + dot(zc, g1c[...])
                    + dot(ze, g1e[...]) + b1[...], 0.0)
    en = dot(h, g2en[...]) + b2en[...]
    hm = jnp.maximum(dot(zr, gnr[...]) + dot(en, gne[...]) + bn1[...], 0.0)
    zout[...] = en + dot(hm, gm2[...]) + bm2[...]


def _edge_mlp_last_body(xr, xc, ea, g1r, g1c, g1e, b1, g2en, b2en, zout):
    zr = xr[...]
    zc = xc[...]
    ze = ea[...]
    dot = functools.partial(jnp.dot, preferred_element_type=jnp.float32)
    h = jnp.maximum(dot(zr, g1r[...]) + dot(zc, g1c[...])
                    + dot(ze, g1e[...]) + b1[...], 0.0)
    zout[...] = dot(h, g2en[...]) + b2en[...]


_BR = 4000


def _edge_call(xr_p, xc_p, ea_p, w, last):
    espec = pl.BlockSpec((_BR, 128), lambda i: (i, 0))
    in_specs = [espec, espec, espec] + [
        pl.BlockSpec(a.shape, lambda i: (0, 0)) for a in w
    ]
    body = _edge_mlp_last_body if last else _edge_mlp_body
    return pl.pallas_call(
        body,
        grid=(EQ // _BR,),
        in_specs=in_specs,
        out_specs=espec,
        out_shape=jax.ShapeDtypeStruct((EQ, 128), jnp.float32),
    )(xr_p, xc_p, ea_p, *w)


def _node_body(xp, s0, s1, c0, c1, gx, gm, b1, g2, b2, xout):
    dot = functools.partial(jnp.dot, preferred_element_type=jnp.float32)
    cnt = jnp.maximum(c0[...] + c1[...], 1.0)
    mean = (s0[...] + s1[...]) / cnt
    h = jnp.maximum(dot(xp[...], gx[...]) + dot(mean, gm[...]) + b1[...], 0.0)
    xout[...] = dot(h, g2[...]) + b2[...]


_BN = 3128


def _node_call(xp, s_parts, c_parts, w):
    nspec = pl.BlockSpec((_BN, 128), lambda i: (i, 0))
    return pl.pallas_call(
        _node_body,
        grid=(NQ // _BN,),
        in_specs=[nspec] * 5 + [
            pl.BlockSpec(a.shape, lambda i: (0, 0)) for a in w
        ],
        out_specs=nspec,
        out_shape=jax.ShapeDtypeStruct((NQ, 128), jnp.float32),
    )(xp, s_parts[0], s_parts[1], c_parts[0], c_parts[1], *w)


def _place(w, r0, c0=0, rows=8, cols=16):
    out = jnp.zeros((rows, cols), jnp.float32)
    return out.at[r0:r0 + w.shape[0], c0:c0 + w.shape[1]].set(w)


def _blk(w):
    return jnp.kron(jnp.eye(16, dtype=jnp.float32), w)


def _edge_weights(lp, dx, dea, last):
    w1 = lp["edge"]["W1"]
    g1r = _blk(_place(w1[0:dx], 0))
    g1c = _blk(_place(w1[dx:2 * dx], 0))
    g1e = _blk(_place(w1[2 * dx:], 0))
    b1 = jnp.tile(lp["edge"]["b1"], 16)[None]
    w2 = lp["edge"]["W2"]
    g2en = _blk(_place(w2, 0, 0, rows=16, cols=8))
    b2en = jnp.tile(_place(lp["edge"]["b2"][None], 0, 0, rows=1, cols=8)[0],
                    16)[None]
    ws = [g1r, g1c, g1e, b1, g2en, b2en]
    if last:
        return ws
    v1 = lp["node1"]["W1"]
    gnr = _blk(_place(v1[0:dx], 0))
    gne = _blk(_place(v1[dx:], 0))
    bn1 = jnp.tile(lp["node1"]["b1"], 16)[None]
    v2 = lp["node1"]["W2"]
    gm2 = _blk(_place(v2, 0, 4, rows=16, cols=8))
    bm2 = jnp.tile(_place(lp["node1"]["b2"][None], 0, 4, rows=1, cols=8)[0],
                   16)[None]
    return ws + [gnr, gne, bn1, gm2, bm2]


def _node_weights(lp, dx):
    u1 = lp["node2"]["W1"]
    gx = _blk(_place(u1[0:dx], 0))
    gm = _blk(_place(u1[dx:], 4))
    b1 = jnp.tile(lp["node2"]["b1"], 16)[None]
    u2 = lp["node2"]["W2"]
    g2 = _blk(_place(u2, 0, 0, rows=16, cols=8))
    b2 = jnp.tile(_place(lp["node2"]["b2"][None], 0, 0, rows=1, cols=8)[0],
                  16)[None]
    return [gx, gm, b1, g2, b2]


def kernel(x, edge_index, edge_attr, params):
    row2d = edge_index[0].reshape(ER, 128)
    col2d = edge_index[1].reshape(ER, 128)
    dx0 = x.shape[1]
    x_tbl = jnp.zeros((N_PAD, 8), jnp.float32).at[:N_NODES, :dx0].set(x)
    dea0 = edge_attr.shape[1]
    ea_p = jnp.pad(edge_attr, ((0, 0), (0, 8 - dea0))).reshape(EQ, 128)
    zeros = jnp.zeros((N_PAD, 8), jnp.float32)

    cnt_parts = None
    layers = params["layers"]
    dx, dea = dx0, dea0
    for li, lp in enumerate(layers):
        last = li == len(layers) - 1
        xr, xc = _gather_kernel()(x_tbl, row2d, col2d)
        xr_p = xr.reshape(EQ, 128)
        xc_p = xc.reshape(EQ, 128)
        ew = _edge_weights(lp, dx, dea, last)
        z_p = _edge_call(xr_p, xc_p, ea_p, ew, last)
        if last:
            ea_p = z_p
            break
        z_flat = z_p.reshape(E, 8)
        if cnt_parts is None:
            ones = jnp.ones((128, 8), jnp.float32)
            s_parts, cnt_parts = _scatter_kernel(True)(z_flat, col2d,
                                                       zeros, ones)
        else:
            s_parts = _scatter_kernel(False)(z_flat, col2d, zeros)
            if isinstance(s_parts, (list, tuple)):
                s_parts = s_parts[0]
        nw = _node_weights(lp, dx)
        xq_new = _node_call(x_tbl.reshape(NQ, 128),
                            s_parts.reshape(2, NQ, 128),
                            cnt_parts.reshape(2, NQ, 128), nw)
        x_tbl = xq_new.reshape(N_PAD, 8)
        ea_p = z_p
        dx, dea = 4, 4

    return ea_p.reshape(E, 8)[:, 0:1]


# speedup vs baseline: 24.3146x; 1.8617x over previous
import functools

import jax
import jax.numpy as jnp
from jax import lax
from jax.experimental import pallas as pl
from jax.experimental.pallas import tpu as pltpu
from jax.experimental.pallas import tpu_sc as plsc

N_NODES = 100000
N_PAD = 100096
NQ = N_PAD // 16
E = 6400000
EQ = E // 16
ER = E // 128
NW = 32
RPW = ER // NW
KC = 22
NCH = RPW // KC
CHE = KC * 128
TAIL0 = NW * RPW
NSEG = N_PAD // 16


@functools.lru_cache(maxsize=None)
def _mesh():
    return plsc.VectorSubcoreMesh(core_axis_name="c", subcore_axis_name="s")


def _wid():
    return lax.axis_index("s") * 2 + lax.axis_index("c")


def _gather_chunk(tbl, idx2d, out_hbm, idx_v, rows_v, sem, row0, nrows):
    pltpu.sync_copy(idx2d.at[pl.ds(row0, nrows), :],
                    idx_v.at[pl.ds(0, nrows), :])
    cps = []
    for j in range(nrows):
        cps.append(pltpu.async_copy(
            tbl.at[idx_v.at[j]],
            rows_v.at[pl.ds(j * 128, 128), :], sem))
    for cp in cps:
        cp.wait()
    pltpu.sync_copy(rows_v.at[pl.ds(0, nrows * 128), :],
                    out_hbm.at[pl.ds(row0 * 128, nrows * 128), :])


def _gather_body(x_tbl, row2d, col2d, xr_out, xc_out,
                 idx_v, rows_v, idx2_v, rows2_v, sem1, sem2):
    w = _wid()
    base = w * RPW

    def step(i, carry):
        r0 = base + i * KC
        _gather_chunk(x_tbl, row2d, xr_out, idx_v, rows_v, sem1, r0, KC)
        _gather_chunk(x_tbl, col2d, xc_out, idx2_v, rows2_v, sem2, r0, KC)
        return carry

    lax.fori_loop(0, NCH, step, 0)

    @pl.when(w < ER - TAIL0)
    def _tail():
        r0 = TAIL0 + w
        _gather_chunk(x_tbl, row2d, xr_out, idx_v, rows_v, sem1, r0, 1)
        _gather_chunk(x_tbl, col2d, xc_out, idx2_v, rows2_v, sem2, r0, 1)


@functools.lru_cache(maxsize=None)
def _gather_kernel():
    return pl.kernel(
        _gather_body,
        out_type=[jax.ShapeDtypeStruct((E, 8), jnp.float32),
                  jax.ShapeDtypeStruct((E, 8), jnp.float32)],
        mesh=_mesh(),
        compiler_params=pltpu.CompilerParams(use_tc_tiling_on_sc=False),
        scratch_types=[
            pltpu.VMEM((KC, 128), jnp.int32),
            pltpu.VMEM((CHE, 8), jnp.float32),
            pltpu.VMEM((KC, 128), jnp.int32),
            pltpu.VMEM((CHE, 8), jnp.float32),
            pltpu.SemaphoreType.DMA,
            pltpu.SemaphoreType.DMA,
        ],
    )


def _scatter_chunk(m_hbm, idx2d, acc_s, cnt_s, idx_v, upd_v, ones_v,
                   row0, nrows, with_counts):
    pltpu.sync_copy(idx2d.at[pl.ds(row0, nrows), :],
                    idx_v.at[pl.ds(0, nrows), :])
    pltpu.sync_copy(m_hbm.at[pl.ds(row0 * 128, nrows * 128), :],
                    upd_v.at[pl.ds(0, nrows * 128), :])
    for j in range(nrows):
        pltpu.sync_copy(upd_v.at[pl.ds(j * 128, 128), :],
                        acc_s.at[idx_v.at[j]], add=True)
        if with_counts:
            pltpu.sync_copy(ones_v, cnt_s.at[idx_v.at[j]], add=True)


def _scatter_body(m_hbm, col2d, zeros_hbm, ones_hbm, out_hbm, cnt_hbm,
                  acc_s, cnt_s, idx_v, upd_v, ones_v, with_counts):
    c = lax.axis_index("c")
    s = lax.axis_index("s")
    w = _wid()
    pltpu.sync_copy(zeros_hbm.at[pl.ds(s * NSEG, NSEG), :],
                    acc_s.at[pl.ds(s * NSEG, NSEG), :])
    if with_counts:
        pltpu.sync_copy(zeros_hbm.at[pl.ds(s * NSEG, NSEG), :],
                        cnt_s.at[pl.ds(s * NSEG, NSEG), :])
        pltpu.sync_copy(ones_hbm, ones_v)
    plsc.subcore_barrier()
    base = w * RPW

    def step(i, carry):
        _scatter_chunk(m_hbm, col2d, acc_s, cnt_s, idx_v, upd_v, ones_v,
                       base + i * KC, KC, with_counts)
        return carry

    lax.fori_loop(0, NCH, step, 0)

    @pl.when(w < ER - TAIL0)
    def _tail():
        _scatter_chunk(m_hbm, col2d, acc_s, cnt_s, idx_v, upd_v, ones_v,
                       TAIL0 + w, 1, with_counts)

    plsc.subcore_barrier()
    pltpu.sync_copy(acc_s.at[pl.ds(s * NSEG, NSEG), :],
                    out_hbm.at[c, pl.ds(s * NSEG, NSEG), :])
    if with_counts:
        pltpu.sync_copy(cnt_s.at[pl.ds(s * NSEG, NSEG), :],
                        cnt_hbm.at[c, pl.ds(s * NSEG, NSEG), :])


@functools.lru_cache(maxsize=None)
def _scatter_kernel(with_counts):
    n_out = 2 if with_counts else 1
    if with_counts:
        def body(m_hbm, col2d, zeros_hbm, ones_hbm, out_hbm, cnt_hbm,
                 acc_s, cnt_s, idx_v, upd_v, ones_v):
            return _scatter_body(m_hbm, col2d, zeros_hbm, ones_hbm,
                                 out_hbm, cnt_hbm,
                                 acc_s, cnt_s, idx_v, upd_v, ones_v, True)
    else:
        def body(m_hbm, col2d, zeros_hbm, out_hbm,
                 acc_s, idx_v, upd_v):
            return _scatter_body(m_hbm, col2d, zeros_hbm, None,
                                 out_hbm, None,
                                 acc_s, None, idx_v, upd_v, None, False)
    out_type = [jax.ShapeDtypeStruct((2, N_PAD, 8), jnp.float32)] * n_out
    scratch = [pltpu.VMEM_SHARED((N_PAD, 8), jnp.float32)]
    if with_counts:
        scratch.append(pltpu.VMEM_SHARED((N_PAD, 8), jnp.float32))
    scratch += [
        pltpu.VMEM((KC, 128), jnp.int32),
        pltpu.VMEM((CHE, 8), jnp.float32),
    ]
    if with_counts:
        scratch.append(pltpu.VMEM((128, 8), jnp.float32))
    return pl.kernel(body, out_type=out_type, mesh=_mesh(),
                     compiler_params=pltpu.CompilerParams(
                         use_tc_tiling_on_sc=False),
                     scratch_types=scratch)


def _pack_chunk(eaT, zeros1d, z0_out, pln_v, zb_v, row0, nrows, first):
    n = nrows * 128
    if first:
        pltpu.sync_copy(zeros1d, zb_v)
    for f in range(3):
        pltpu.sync_copy(eaT.at[f, pl.ds(row0 * 128, n)],
                        pln_v.at[f, pl.ds(0, n)])
    lanes = lax.iota(jnp.int32, 16) * 8

    def grp(g, carry):
        base8 = g * 128
        for f in range(3):
            v = pln_v[f, pl.ds(g * 16, 16)]
            plsc.store_scatter(zb_v, [lanes + (base8 + f)], v)
        return carry

    lax.fori_loop(0, nrows * 8, grp, 0)
    pltpu.sync_copy(zb_v.at[pl.ds(0, n * 8)],
                    z0_out.at[pl.ds(row0 * 1024, n * 8)])


def _pack_body(eaT, zeros1d, z0_out, pln_v, zb_v):
    w = _wid()
    base = w * RPW

    def step(i, carry):
        _pack_chunk(eaT, zeros1d, z0_out, pln_v, zb_v,
                    base + i * KC, KC, False)
        return carry

    _pack_chunk(eaT, zeros1d, z0_out, pln_v, zb_v, base, KC, True)
    lax.fori_loop(1, NCH, step, 0)

    @pl.when(w < ER - TAIL0)
    def _tail():
        _pack_chunk(eaT, zeros1d, z0_out, pln_v, zb_v,
                    TAIL0 + w, 1, False)


@functools.lru_cache(maxsize=None)
def _pack_kernel():
    return pl.kernel(
        _pack_body,
        out_type=jax.ShapeDtypeStruct((E * 8,), jnp.float32),
        mesh=_mesh(),
        compiler_params=pltpu.CompilerParams(use_tc_tiling_on_sc=False,
                                             needs_layout_passes=False),
        scratch_types=[
            pltpu.VMEM((3, CHE), jnp.float32),
            pltpu.VMEM((CHE * 8,), jnp.float32),
        ],
    )


def _edge_mlp_body(xr, xc, ea, g1r, g1c, g1e, b1, g2en, b2en,
                   gnr, gne, bn1, gm2, bm2, zout):
    zr = xr[...]
    zc = xc[...]
    ze = ea[...]
    dot = functools.partial(jnp.dot, preferred_element_type=jnp.float32)
    h = jnp.maximum(dot(zr, g1r[...]) + dot(zc, g1c[...])
                    + dot(ze, g1e[...]) + b1[...], 0.0)
    en = dot(h, g2en[...]) + b2en[...]
    hm = jnp.maximum(dot(zr, gnr[...]) + dot(en, gne[...]) + bn1[...], 0.0)
    zout[...] = en + dot(hm, gm2[...]) + bm2[...]


def _edge_mlp_last_body(xr, xc, ea, g1r, g1c, g1e, b1, g2en, b2en, zout):
    zr = xr[...]
    zc = xc[...]
    ze = ea[...]
    dot = functools.partial(jnp.dot, preferred_element_type=jnp.float32)
    h = jnp.maximum(dot(zr, g1r[...]) + dot(zc, g1c[...])
                    + dot(ze, g1e[...]) + b1[...], 0.0)
    zout[...] = dot(h, g2en[...]) + b2en[...]


_BR = 4000


def _edge_call(xr_p, xc_p, ea_p, w, last):
    espec = pl.BlockSpec((_BR, 128), lambda i: (i, 0))
    in_specs = [espec, espec, espec] + [
        pl.BlockSpec(a.shape, lambda i: (0, 0)) for a in w
    ]
    body = _edge_mlp_last_body if last else _edge_mlp_body
    return pl.pallas_call(
        body,
        grid=(EQ // _BR,),
        in_specs=in_specs,
        out_specs=espec,
        out_shape=jax.ShapeDtypeStruct((EQ, 128), jnp.float32),
    )(xr_p, xc_p, ea_p, *w)


def _node_body(xp, s0, s1, c0, c1, gx, gm, b1, g2, b2, xout):
    dot = functools.partial(jnp.dot, preferred_element_type=jnp.float32)
    cnt = jnp.maximum(c0[...] + c1[...], 1.0)
    mean = (s0[...] + s1[...]) / cnt
    h = jnp.maximum(dot(xp[...], gx[...]) + dot(mean, gm[...]) + b1[...], 0.0)
    xout[...] = dot(h, g2[...]) + b2[...]


_BN = 3128


def _node_call(xp, s_parts, c_parts, w):
    nspec = pl.BlockSpec((_BN, 128), lambda i: (i, 0))
    return pl.pallas_call(
        _node_body,
        grid=(NQ // _BN,),
        in_specs=[nspec] * 5 + [
            pl.BlockSpec(a.shape, lambda i: (0, 0)) for a in w
        ],
        out_specs=nspec,
        out_shape=jax.ShapeDtypeStruct((NQ, 128), jnp.float32),
    )(xp, s_parts[0], s_parts[1], c_parts[0], c_parts[1], *w)


def _place(w, r0, c0=0, rows=8, cols=16):
    out = jnp.zeros((rows, cols), jnp.float32)
    return out.at[r0:r0 + w.shape[0], c0:c0 + w.shape[1]].set(w)


def _blk(w):
    return jnp.kron(jnp.eye(16, dtype=jnp.float32), w)


def _edge_weights(lp, dx, dea, last):
    w1 = lp["edge"]["W1"]
    g1r = _blk(_place(w1[0:dx], 0))
    g1c = _blk(_place(w1[dx:2 * dx], 0))
    g1e = _blk(_place(w1[2 * dx:], 0))
    b1 = jnp.tile(lp["edge"]["b1"], 16)[None]
    w2 = lp["edge"]["W2"]
    g2en = _blk(_place(w2, 0, 0, rows=16, cols=8))
    b2en = jnp.tile(_place(lp["edge"]["b2"][None], 0, 0, rows=1, cols=8)[0],
                    16)[None]
    ws = [g1r, g1c, g1e, b1, g2en, b2en]
    if last:
        return ws
    v1 = lp["node1"]["W1"]
    gnr = _blk(_place(v1[0:dx], 0))
    gne = _blk(_place(v1[dx:], 0))
    bn1 = jnp.tile(lp["node1"]["b1"], 16)[None]
    v2 = lp["node1"]["W2"]
    gm2 = _blk(_place(v2, 0, 4, rows=16, cols=8))
    bm2 = jnp.tile(_place(lp["node1"]["b2"][None], 0, 4, rows=1, cols=8)[0],
                   16)[None]
    return ws + [gnr, gne, bn1, gm2, bm2]


def _node_weights(lp, dx):
    u1 = lp["node2"]["W1"]
    gx = _blk(_place(u1[0:dx], 0))
    gm = _blk(_place(u1[dx:], 4))
    b1 = jnp.tile(lp["node2"]["b1"], 16)[None]
    u2 = lp["node2"]["W2"]
    g2 = _blk(_place(u2, 0, 0, rows=16, cols=8))
    b2 = jnp.tile(_place(lp["node2"]["b2"][None], 0, 0, rows=1, cols=8)[0],
                  16)[None]
    return [gx, gm, b1, g2, b2]


def kernel(x, edge_index, edge_attr, params):
    row2d = edge_index[0].reshape(ER, 128)
    col2d = edge_index[1].reshape(ER, 128)
    dx0 = x.shape[1]
    x_tbl = jnp.zeros((N_PAD, 8), jnp.float32).at[:N_NODES, :dx0].set(x)
    dea0 = edge_attr.shape[1]
    eaT = jnp.pad(edge_attr.T, ((0, 8 - dea0), (0, 0)))
    zeros1d = jnp.zeros((CHE * 8,), jnp.float32)
    ea_p = _pack_kernel()(eaT, zeros1d).reshape(EQ, 128)
    zeros = jnp.zeros((N_PAD, 8), jnp.float32)

    cnt_parts = None
    layers = params["layers"]
    dx, dea = dx0, dea0
    for li, lp in enumerate(layers):
        last = li == len(layers) - 1
        xr, xc = _gather_kernel()(x_tbl, row2d, col2d)
        xr_p = xr.reshape(EQ, 128)
        xc_p = xc.reshape(EQ, 128)
        ew = _edge_weights(lp, dx, dea, last)
        z_p = _edge_call(xr_p, xc_p, ea_p, ew, last)
        if last:
            ea_p = z_p
            break
        z_flat = z_p.reshape(E, 8)
        if cnt_parts is None:
            ones = jnp.ones((128, 8), jnp.float32)
            s_parts, cnt_parts = _scatter_kernel(True)(z_flat, col2d,
                                                       zeros, ones)
        else:
            s_parts = _scatter_kernel(False)(z_flat, col2d, zeros)
            if isinstance(s_parts, (list, tuple)):
                s_parts = s_parts[0]
        nw = _node_weights(lp, dx)
        xq_new = _node_call(x_tbl.reshape(NQ, 128),
                            s_parts.reshape(2, NQ, 128),
                            cnt_parts.reshape(2, NQ, 128), nw)
        x_tbl = xq_new.reshape(N_PAD, 8)
        ea_p = z_p
        dx, dea = 4, 4

    return ea_p.reshape(E, 8)[:, 0:1]


# speedup vs baseline: 42.1325x; 1.7328x over previous
import functools

import jax
import jax.numpy as jnp
from jax import lax
from jax.experimental import pallas as pl
from jax.experimental.pallas import tpu as pltpu
from jax.experimental.pallas import tpu_sc as plsc

N_NODES = 100000
N_PAD = 100096
NQ = N_PAD // 16
E = 6400000
EQ = E // 16
ER = E // 128
NW = 32
RPW = ER // NW
KC = 22
NCH = RPW // KC
CHE = KC * 128
TAIL0 = NW * RPW
NSEG = N_PAD // 16


@functools.lru_cache(maxsize=None)
def _mesh():
    return plsc.VectorSubcoreMesh(core_axis_name="c", subcore_axis_name="s")


def _wid():
    return lax.axis_index("s") * 2 + lax.axis_index("c")


def _gather_chunk(tbl, idx2d, out_hbm, idx_v, rows_v, sem, row0, nrows):
    pltpu.sync_copy(idx2d.at[pl.ds(row0, nrows), :],
                    idx_v.at[pl.ds(0, nrows), :])
    cps = []
    for j in range(nrows):
        cps.append(pltpu.async_copy(
            tbl.at[idx_v.at[j]],
            rows_v.at[pl.ds(j * 128, 128), :], sem))
    for cp in cps:
        cp.wait()
    pltpu.sync_copy(rows_v.at[pl.ds(0, nrows * 128), :],
                    out_hbm.at[pl.ds(row0 * 128, nrows * 128), :])


def _gather_body(x_tbl, row2d, col2d, xr_out, xc_out,
                 tbl_s, idx_v, rows_v, idx2_v, rows2_v, sem1, sem2):
    s = lax.axis_index("s")
    w = _wid()
    pltpu.sync_copy(x_tbl.at[pl.ds(s * NSEG, NSEG), :],
                    tbl_s.at[pl.ds(s * NSEG, NSEG), :])
    plsc.subcore_barrier()
    base = w * RPW

    def step(i, carry):
        r0 = base + i * KC
        _gather_chunk(tbl_s, row2d, xr_out, idx_v, rows_v, sem1, r0, KC)
        _gather_chunk(tbl_s, col2d, xc_out, idx2_v, rows2_v, sem2, r0, KC)
        return carry

    lax.fori_loop(0, NCH, step, 0)

    @pl.when(w < ER - TAIL0)
    def _tail():
        r0 = TAIL0 + w
        _gather_chunk(tbl_s, row2d, xr_out, idx_v, rows_v, sem1, r0, 1)
        _gather_chunk(tbl_s, col2d, xc_out, idx2_v, rows2_v, sem2, r0, 1)


@functools.lru_cache(maxsize=None)
def _gather_kernel():
    return pl.kernel(
        _gather_body,
        out_type=[jax.ShapeDtypeStruct((E, 8), jnp.float32),
                  jax.ShapeDtypeStruct((E, 8), jnp.float32)],
        mesh=_mesh(),
        compiler_params=pltpu.CompilerParams(use_tc_tiling_on_sc=False),
        scratch_types=[
            pltpu.VMEM_SHARED((N_PAD, 8), jnp.float32),
            pltpu.VMEM((KC, 128), jnp.int32),
            pltpu.VMEM((CHE, 8), jnp.float32),
            pltpu.VMEM((KC, 128), jnp.int32),
            pltpu.VMEM((CHE, 8), jnp.float32),
            pltpu.SemaphoreType.DMA,
            pltpu.SemaphoreType.DMA,
        ],
    )


def _scatter_chunk(m_hbm, idx2d, acc_s, cnt_s, idx_v, upd_v, ones_v,
                   row0, nrows, with_counts):
    pltpu.sync_copy(idx2d.at[pl.ds(row0, nrows), :],
                    idx_v.at[pl.ds(0, nrows), :])
    pltpu.sync_copy(m_hbm.at[pl.ds(row0 * 128, nrows * 128), :],
                    upd_v.at[pl.ds(0, nrows * 128), :])
    for j in range(nrows):
        pltpu.sync_copy(upd_v.at[pl.ds(j * 128, 128), :],
                        acc_s.at[idx_v.at[j]], add=True)
        if with_counts:
            pltpu.sync_copy(ones_v, cnt_s.at[idx_v.at[j]], add=True)


def _scatter_body(m_hbm, col2d, zeros_hbm, ones_hbm, out_hbm, cnt_hbm,
                  acc_s, cnt_s, idx_v, upd_v, ones_v, with_counts):
    c = lax.axis_index("c")
    s = lax.axis_index("s")
    w = _wid()
    pltpu.sync_copy(zeros_hbm.at[pl.ds(s * NSEG, NSEG), :],
                    acc_s.at[pl.ds(s * NSEG, NSEG), :])
    if with_counts:
        pltpu.sync_copy(zeros_hbm.at[pl.ds(s * NSEG, NSEG), :],
                        cnt_s.at[pl.ds(s * NSEG, NSEG), :])
        pltpu.sync_copy(ones_hbm, ones_v)
    plsc.subcore_barrier()
    base = w * RPW

    def step(i, carry):
        _scatter_chunk(m_hbm, col2d, acc_s, cnt_s, idx_v, upd_v, ones_v,
                       base + i * KC, KC, with_counts)
        return carry

    lax.fori_loop(0, NCH, step, 0)

    @pl.when(w < ER - TAIL0)
    def _tail():
        _scatter_chunk(m_hbm, col2d, acc_s, cnt_s, idx_v, upd_v, ones_v,
                       TAIL0 + w, 1, with_counts)

    plsc.subcore_barrier()
    pltpu.sync_copy(acc_s.at[pl.ds(s * NSEG, NSEG), :],
                    out_hbm.at[c, pl.ds(s * NSEG, NSEG), :])
    if with_counts:
        pltpu.sync_copy(cnt_s.at[pl.ds(s * NSEG, NSEG), :],
                        cnt_hbm.at[c, pl.ds(s * NSEG, NSEG), :])


@functools.lru_cache(maxsize=None)
def _scatter_kernel(with_counts):
    n_out = 2 if with_counts else 1
    if with_counts:
        def body(m_hbm, col2d, zeros_hbm, ones_hbm, out_hbm, cnt_hbm,
                 acc_s, cnt_s, idx_v, upd_v, ones_v):
            return _scatter_body(m_hbm, col2d, zeros_hbm, ones_hbm,
                                 out_hbm, cnt_hbm,
                                 acc_s, cnt_s, idx_v, upd_v, ones_v, True)
    else:
        def body(m_hbm, col2d, zeros_hbm, out_hbm,
                 acc_s, idx_v, upd_v):
            return _scatter_body(m_hbm, col2d, zeros_hbm, None,
                                 out_hbm, None,
                                 acc_s, None, idx_v, upd_v, None, False)
    out_type = [jax.ShapeDtypeStruct((2, N_PAD, 8), jnp.float32)] * n_out
    scratch = [pltpu.VMEM_SHARED((N_PAD, 8), jnp.float32)]
    if with_counts:
        scratch.append(pltpu.VMEM_SHARED((N_PAD, 8), jnp.float32))
    scratch += [
        pltpu.VMEM((KC, 128), jnp.int32),
        pltpu.VMEM((CHE, 8), jnp.float32),
    ]
    if with_counts:
        scratch.append(pltpu.VMEM((128, 8), jnp.float32))
    return pl.kernel(body, out_type=out_type, mesh=_mesh(),
                     compiler_params=pltpu.CompilerParams(
                         use_tc_tiling_on_sc=False),
                     scratch_types=scratch)


def _pack_chunk(eaT, zeros1d, z0_out, pln_v, zb_v, row0, nrows, first):
    n = nrows * 128
    if first:
        pltpu.sync_copy(zeros1d, zb_v)
    for f in range(3):
        pltpu.sync_copy(eaT.at[f, pl.ds(row0 * 128, n)],
                        pln_v.at[f, pl.ds(0, n)])
    lanes = lax.iota(jnp.int32, 16) * 8

    def grp(g, carry):
        base8 = g * 128
        for f in range(3):
            v = pln_v[f, pl.ds(g * 16, 16)]
            plsc.store_scatter(zb_v, [lanes + (base8 + f)], v)
        return carry

    lax.fori_loop(0, nrows * 8, grp, 0)
    pltpu.sync_copy(zb_v.at[pl.ds(0, n * 8)],
                    z0_out.at[pl.ds(row0 * 1024, n * 8)])


def _pack_body(eaT, zeros1d, z0_out, pln_v, zb_v):
    w = _wid()
    base = w * RPW

    def step(i, carry):
        _pack_chunk(eaT, zeros1d, z0_out, pln_v, zb_v,
                    base + i * KC, KC, False)
        return carry

    _pack_chunk(eaT, zeros1d, z0_out, pln_v, zb_v, base, KC, True)
    lax.fori_loop(1, NCH, step, 0)

    @pl.when(w < ER - TAIL0)
    def _tail():
        _pack_chunk(eaT, zeros1d, z0_out, pln_v, zb_v,
                    TAIL0 + w, 1, False)


@functools.lru_cache(maxsize=None)
def _pack_kernel():
    return pl.kernel(
        _pack_body,
        out_type=jax.ShapeDtypeStruct((E * 8,), jnp.float32),
        mesh=_mesh(),
        compiler_params=pltpu.CompilerParams(use_tc_tiling_on_sc=False,
                                             needs_layout_passes=False),
        scratch_types=[
            pltpu.VMEM((3, CHE), jnp.float32),
            pltpu.VMEM((CHE * 8,), jnp.float32),
        ],
    )


def _extract_chunk(z_hbm, out1d, zb_v, o_v, row0, nrows):
    n = nrows * 128
    pltpu.sync_copy(z_hbm.at[pl.ds(row0 * 1024, n * 8)],
                    zb_v.at[pl.ds(0, n * 8)])
    lanes = lax.iota(jnp.int32, 16) * 8

    def grp(g, carry):
        v = plsc.load_gather(zb_v, [lanes + g * 128])
        o_v[pl.ds(g * 16, 16)] = v
        return carry

    lax.fori_loop(0, nrows * 8, grp, 0)
    pltpu.sync_copy(o_v.at[pl.ds(0, n)], out1d.at[pl.ds(row0 * 128, n)])


def _extract_body(z_hbm, out1d, zb_v, o_v):
    w = _wid()
    base = w * RPW

    def step(i, carry):
        _extract_chunk(z_hbm, out1d, zb_v, o_v, base + i * KC, KC)
        return carry

    lax.fori_loop(0, NCH, step, 0)

    @pl.when(w < ER - TAIL0)
    def _tail():
        _extract_chunk(z_hbm, out1d, zb_v, o_v, TAIL0 + w, 1)


@functools.lru_cache(maxsize=None)
def _extract_kernel():
    return pl.kernel(
        _extract_body,
        out_type=jax.ShapeDtypeStruct((E,), jnp.float32),
        mesh=_mesh(),
        compiler_params=pltpu.CompilerParams(use_tc_tiling_on_sc=False,
                                             needs_layout_passes=False),
        scratch_types=[
            pltpu.VMEM((CHE * 8,), jnp.float32),
            pltpu.VMEM((CHE,), jnp.float32),
        ],
    )


def _edge_mlp_body(xr, xc, ea, g1r, g1c, g1e, b1, g2en, b2en,
                   gnr, gne, bn1, gm2, bm2, zout):
    zr = xr[...]
    zc = xc[...]
    ze = ea[...]
    dot = functools.partial(jnp.dot, preferred_element_type=jnp.float32)
    h = jnp.maximum(dot(zr, g1r[...]) + dot(zc, g1c[...])
                    + dot(ze, g1e[...]) + b1[...], 0.0)
    en = dot(h, g2en[...]) + b2en[...]
    hm = jnp.maximum(dot(zr, gnr[...]) + dot(en, gne[...]) + bn1[...], 0.0)
    zout[...] = en + dot(hm, gm2[...]) + bm2[...]


def _edge_mlp_last_body(xr, xc, ea, g1r, g1c, g1e, b1, g2en, b2en, zout):
    zr = xr[...]
    zc = xc[...]
    ze = ea[...]
    dot = functools.partial(jnp.dot, preferred_element_type=jnp.float32)
    h = jnp.maximum(dot(zr, g1r[...]) + dot(zc, g1c[...])
                    + dot(ze, g1e[...]) + b1[...], 0.0)
    zout[...] = dot(h, g2en[...]) + b2en[...]


_BR = 4000


def _edge_call(xr_p, xc_p, ea_p, w, last):
    espec = pl.BlockSpec((_BR, 128), lambda i: (i, 0))
    in_specs = [espec, espec, espec] + [
        pl.BlockSpec(a.shape, lambda i: (0, 0)) for a in w
    ]
    body = _edge_mlp_last_body if last else _edge_mlp_body
    return pl.pallas_call(
        body,
        grid=(EQ // _BR,),
        in_specs=in_specs,
        out_specs=espec,
        out_shape=jax.ShapeDtypeStruct((EQ, 128), jnp.float32),
    )(xr_p, xc_p, ea_p, *w)


def _node_body(xp, s0, s1, c0, c1, gx, gm, b1, g2, b2, xout):
    dot = functools.partial(jnp.dot, preferred_element_type=jnp.float32)
    cnt = jnp.maximum(c0[...] + c1[...], 1.0)
    mean = (s0[...] + s1[...]) / cnt
    h = jnp.maximum(dot(xp[...], gx[...]) + dot(mean, gm[...]) + b1[...], 0.0)
    xout[...] = dot(h, g2[...]) + b2[...]


_BN = 3128


def _node_call(xp, s_parts, c_parts, w):
    nspec = pl.BlockSpec((_BN, 128), lambda i: (i, 0))
    return pl.pallas_call(
        _node_body,
        grid=(NQ // _BN,),
        in_specs=[nspec] * 5 + [
            pl.BlockSpec(a.shape, lambda i: (0, 0)) for a in w
        ],
        out_specs=nspec,
        out_shape=jax.ShapeDtypeStruct((NQ, 128), jnp.float32),
    )(xp, s_parts[0], s_parts[1], c_parts[0], c_parts[1], *w)


def _place(w, r0, c0=0, rows=8, cols=16):
    out = jnp.zeros((rows, cols), jnp.float32)
    return out.at[r0:r0 + w.shape[0], c0:c0 + w.shape[1]].set(w)


def _blk(w):
    return jnp.kron(jnp.eye(16, dtype=jnp.float32), w)


def _edge_weights(lp, dx, dea, last):
    w1 = lp["edge"]["W1"]
    g1r = _blk(_place(w1[0:dx], 0))
    g1c = _blk(_place(w1[dx:2 * dx], 0))
    g1e = _blk(_place(w1[2 * dx:], 0))
    b1 = jnp.tile(lp["edge"]["b1"], 16)[None]
    w2 = lp["edge"]["W2"]
    g2en = _blk(_place(w2, 0, 0, rows=16, cols=8))
    b2en = jnp.tile(_place(lp["edge"]["b2"][None], 0, 0, rows=1, cols=8)[0],
                    16)[None]
    ws = [g1r, g1c, g1e, b1, g2en, b2en]
    if last:
        return ws
    v1 = lp["node1"]["W1"]
    gnr = _blk(_place(v1[0:dx], 0))
    gne = _blk(_place(v1[dx:], 0))
    bn1 = jnp.tile(lp["node1"]["b1"], 16)[None]
    v2 = lp["node1"]["W2"]
    gm2 = _blk(_place(v2, 0, 4, rows=16, cols=8))
    bm2 = jnp.tile(_place(lp["node1"]["b2"][None], 0, 4, rows=1, cols=8)[0],
                   16)[None]
    return ws + [gnr, gne, bn1, gm2, bm2]


def _node_weights(lp, dx):
    u1 = lp["node2"]["W1"]
    gx = _blk(_place(u1[0:dx], 0))
    gm = _blk(_place(u1[dx:], 4))
    b1 = jnp.tile(lp["node2"]["b1"], 16)[None]
    u2 = lp["node2"]["W2"]
    g2 = _blk(_place(u2, 0, 0, rows=16, cols=8))
    b2 = jnp.tile(_place(lp["node2"]["b2"][None], 0, 0, rows=1, cols=8)[0],
                  16)[None]
    return [gx, gm, b1, g2, b2]


def kernel(x, edge_index, edge_attr, params):
    row2d = edge_index[0].reshape(ER, 128)
    col2d = edge_index[1].reshape(ER, 128)
    dx0 = x.shape[1]
    x_tbl = jnp.zeros((N_PAD, 8), jnp.float32).at[:N_NODES, :dx0].set(x)
    dea0 = edge_attr.shape[1]
    eaT = jnp.pad(edge_attr.T, ((0, 8 - dea0), (0, 0)))
    zeros1d = jnp.zeros((CHE * 8,), jnp.float32)
    ea_p = _pack_kernel()(eaT, zeros1d).reshape(EQ, 128)
    zeros = jnp.zeros((N_PAD, 8), jnp.float32)

    cnt_parts = None
    layers = params["layers"]
    dx, dea = dx0, dea0
    for li, lp in enumerate(layers):
        last = li == len(layers) - 1
        xr, xc = _gather_kernel()(x_tbl, row2d, col2d)
        xr_p = xr.reshape(EQ, 128)
        xc_p = xc.reshape(EQ, 128)
        ew = _edge_weights(lp, dx, dea, last)
        z_p = _edge_call(xr_p, xc_p, ea_p, ew, last)
        if last:
            ea_p = z_p
            break
        z_flat = z_p.reshape(E, 8)
        if cnt_parts is None:
            ones = jnp.ones((128, 8), jnp.float32)
            s_parts, cnt_parts = _scatter_kernel(True)(z_flat, col2d,
                                                       zeros, ones)
        else:
            s_parts = _scatter_kernel(False)(z_flat, col2d, zeros)
            if isinstance(s_parts, (list, tuple)):
                s_parts = s_parts[0]
        nw = _node_weights(lp, dx)
        xq_new = _node_call(x_tbl.reshape(NQ, 128),
                            s_parts.reshape(2, NQ, 128),
                            cnt_parts.reshape(2, NQ, 128), nw)
        x_tbl = xq_new.reshape(N_PAD, 8)
        ea_p = z_p
        dx, dea = 4, 4

    return _extract_kernel()(ea_p.reshape(E * 8)).reshape(E, 1)


# speedup vs baseline: 44.5175x; 1.0566x over previous
import functools

import jax
import jax.numpy as jnp
from jax import lax
from jax.experimental import pallas as pl
from jax.experimental.pallas import tpu as pltpu
from jax.experimental.pallas import tpu_sc as plsc

N_NODES = 100000
N_PAD = 100096
NQ = N_PAD // 16
E = 6400000
EH = E // 2
EQH = EH // 16
ER = E // 128
ERH = ER // 2
NW = 32
RPW = ERH // NW
KC = 11
NCH = RPW // KC
CHE = KC * 128
TAILN = ERH - NW * RPW
NSEG = N_PAD // 16


@functools.lru_cache(maxsize=None)
def _mesh():
    return plsc.VectorSubcoreMesh(core_axis_name="c", subcore_axis_name="s")


def _wid():
    return lax.axis_index("s") * 2 + lax.axis_index("c")


def _sc_params(**kw):
    return pltpu.CompilerParams(use_tc_tiling_on_sc=False, **kw)


def _half_loop(h, fn, tail_fn):
    w = _wid()
    base_loc = w * RPW

    def step(i, carry):
        r_loc = base_loc + i * KC
        fn(h * ERH + r_loc, r_loc)
        return carry

    lax.fori_loop(0, NCH, step, 0)

    @pl.when(w < TAILN)
    def _tail():
        r_loc = NW * RPW + w
        tail_fn(h * ERH + r_loc, r_loc)


def _gather_chunk(tbl, idx2d, out_hbm, idx_v, rows_v, sem,
                  r_abs, r_loc, nrows):
    pltpu.sync_copy(idx2d.at[pl.ds(r_abs, nrows), :],
                    idx_v.at[pl.ds(0, nrows), :])
    cps = []
    for j in range(nrows):
        cps.append(pltpu.async_copy(
            tbl.at[idx_v.at[j]],
            rows_v.at[pl.ds(j * 128, 128), :], sem))
    for cp in cps:
        cp.wait()
    pltpu.sync_copy(rows_v.at[pl.ds(0, nrows * 128), :],
                    out_hbm.at[pl.ds(r_loc * 128, nrows * 128), :])


def _gather_body(x_tbl, row2d, col2d, xr_out, xc_out,
                 tbl_s, idx_v, rows_v, idx2_v, rows2_v, sem1, sem2, h):
    s = lax.axis_index("s")
    pltpu.sync_copy(x_tbl.at[pl.ds(s * NSEG, NSEG), :],
                    tbl_s.at[pl.ds(s * NSEG, NSEG), :])
    plsc.subcore_barrier()

    def chunk(r_abs, r_loc, nrows):
        _gather_chunk(tbl_s, row2d, xr_out, idx_v, rows_v, sem1,
                      r_abs, r_loc, nrows)
        _gather_chunk(tbl_s, col2d, xc_out, idx2_v, rows2_v, sem2,
                      r_abs, r_loc, nrows)

    _half_loop(h,
               lambda ra, rl: chunk(ra, rl, KC),
               lambda ra, rl: chunk(ra, rl, 1))


@functools.lru_cache(maxsize=None)
def _gather_kernel(h):
    return pl.kernel(
        functools.partial(_gather_body, h=h),
        out_type=[jax.ShapeDtypeStruct((EH, 8), jnp.float32),
                  jax.ShapeDtypeStruct((EH, 8), jnp.float32)],
        mesh=_mesh(),
        compiler_params=_sc_params(),
        scratch_types=[
            pltpu.VMEM_SHARED((N_PAD, 8), jnp.float32),
            pltpu.VMEM((KC, 128), jnp.int32),
            pltpu.VMEM((CHE, 8), jnp.float32),
            pltpu.VMEM((KC, 128), jnp.int32),
            pltpu.VMEM((CHE, 8), jnp.float32),
            pltpu.SemaphoreType.DMA,
            pltpu.SemaphoreType.DMA,
        ],
    )


def _scatter_chunk(m_hbm, idx2d, acc_s, cnt_s, idx_v, upd_v, ones_v,
                   r_abs, r_loc, nrows, with_counts):
    pltpu.sync_copy(idx2d.at[pl.ds(r_abs, nrows), :],
                    idx_v.at[pl.ds(0, nrows), :])
    pltpu.sync_copy(m_hbm.at[pl.ds(r_loc * 128, nrows * 128), :],
                    upd_v.at[pl.ds(0, nrows * 128), :])
    for j in range(nrows):
        pltpu.sync_copy(upd_v.at[pl.ds(j * 128, 128), :],
                        acc_s.at[idx_v.at[j]], add=True)
        if with_counts:
            pltpu.sync_copy(ones_v, cnt_s.at[idx_v.at[j]], add=True)


def _scatter_body(m_hbm, col2d, zeros_hbm, ones_hbm, out_hbm, cnt_hbm,
                  acc_s, cnt_s, idx_v, upd_v, ones_v, h, with_counts):
    c = lax.axis_index("c")
    s = lax.axis_index("s")
    pltpu.sync_copy(zeros_hbm.at[pl.ds(s * NSEG, NSEG), :],
                    acc_s.at[pl.ds(s * NSEG, NSEG), :])
    if with_counts:
        pltpu.sync_copy(zeros_hbm.at[pl.ds(s * NSEG, NSEG), :],
                        cnt_s.at[pl.ds(s * NSEG, NSEG), :])
        pltpu.sync_copy(ones_hbm, ones_v)
    plsc.subcore_barrier()

    def chunk(r_abs, r_loc, nrows):
        _scatter_chunk(m_hbm, col2d, acc_s, cnt_s, idx_v, upd_v, ones_v,
                       r_abs, r_loc, nrows, with_counts)

    _half_loop(h,
               lambda ra, rl: chunk(ra, rl, KC),
               lambda ra, rl: chunk(ra, rl, 1))
    plsc.subcore_barrier()
    pltpu.sync_copy(acc_s.at[pl.ds(s * NSEG, NSEG), :],
                    out_hbm.at[c, pl.ds(s * NSEG, NSEG), :])
    if with_counts:
        pltpu.sync_copy(cnt_s.at[pl.ds(s * NSEG, NSEG), :],
                        cnt_hbm.at[c, pl.ds(s * NSEG, NSEG), :])


@functools.lru_cache(maxsize=None)
def _scatter_kernel(h, with_counts):
    n_out = 2 if with_counts else 1
    if with_counts:
        def body(m_hbm, col2d, zeros_hbm, ones_hbm, out_hbm, cnt_hbm,
                 acc_s, cnt_s, idx_v, upd_v, ones_v):
            return _scatter_body(m_hbm, col2d, zeros_hbm, ones_hbm,
                                 out_hbm, cnt_hbm,
                                 acc_s, cnt_s, idx_v, upd_v, ones_v,
                                 h, True)
    else:
        def body(m_hbm, col2d, zeros_hbm, out_hbm,
                 acc_s, idx_v, upd_v):
            return _scatter_body(m_hbm, col2d, zeros_hbm, None,
                                 out_hbm, None,
                                 acc_s, None, idx_v, upd_v, None,
                                 h, False)
    out_type = [jax.ShapeDtypeStruct((2, N_PAD, 8), jnp.float32)] * n_out
    scratch = [pltpu.VMEM_SHARED((N_PAD, 8), jnp.float32)]
    if with_counts:
        scratch.append(pltpu.VMEM_SHARED((N_PAD, 8), jnp.float32))
    scratch += [
        pltpu.VMEM((KC, 128), jnp.int32),
        pltpu.VMEM((CHE, 8), jnp.float32),
    ]
    if with_counts:
        scratch.append(pltpu.VMEM((128, 8), jnp.float32))
    return pl.kernel(body, out_type=out_type, mesh=_mesh(),
                     compiler_params=_sc_params(),
                     scratch_types=scratch)


def _pack_chunk(eaT, z0_out, pln_v, zb_v, r_abs, r_loc, nrows):
    n = nrows * 128
    for f in range(3):
        pltpu.sync_copy(eaT.at[f, pl.ds(r_abs * 128, n)],
                        pln_v.at[f, pl.ds(0, n)])
    lanes = lax.iota(jnp.int32, 16) * 8

    def grp(g, carry):
        base8 = g * 128
        for f in range(3):
            v = pln_v[f, pl.ds(g * 16, 16)]
            plsc.store_scatter(zb_v, [lanes + (base8 + f)], v)
        return carry

    lax.fori_loop(0, nrows * 8, grp, 0)
    pltpu.sync_copy(zb_v.at[pl.ds(0, n * 8)],
                    z0_out.at[pl.ds(r_loc * 1024, n * 8)])


def _pack_body(eaT, zeros1d, z0_out, pln_v, zb_v, h):
    pltpu.sync_copy(zeros1d, zb_v)
    _half_loop(h,
               lambda ra, rl: _pack_chunk(eaT, z0_out, pln_v, zb_v,
                                          ra, rl, KC),
               lambda ra, rl: _pack_chunk(eaT, z0_out, pln_v, zb_v,
                                          ra, rl, 1))


@functools.lru_cache(maxsize=None)
def _pack_kernel(h):
    return pl.kernel(
        functools.partial(_pack_body, h=h),
        out_type=jax.ShapeDtypeStruct((EH * 8,), jnp.float32),
        mesh=_mesh(),
        compiler_params=_sc_params(needs_layout_passes=False),
        scratch_types=[
            pltpu.VMEM((3, CHE), jnp.float32),
            pltpu.VMEM((CHE * 8,), jnp.float32),
        ],
    )


def _extract_chunk(z_hbm, out1d, zb_v, o_v, r_loc, nrows):
    n = nrows * 128
    pltpu.sync_copy(z_hbm.at[pl.ds(r_loc * 1024, n * 8)],
                    zb_v.at[pl.ds(0, n * 8)])
    lanes = lax.iota(jnp.int32, 16) * 8

    def grp(g, carry):
        v = plsc.load_gather(zb_v, [lanes + g * 128])
        o_v[pl.ds(g * 16, 16)] = v
        return carry

    lax.fori_loop(0, nrows * 8, grp, 0)
    pltpu.sync_copy(o_v.at[pl.ds(0, n)], out1d.at[pl.ds(r_loc * 128, n)])


def _extract_body(z_hbm, out1d, zb_v, o_v, h):
    _half_loop(h,
               lambda ra, rl: _extract_chunk(z_hbm, out1d, zb_v, o_v,
                                             rl, KC),
               lambda ra, rl: _extract_chunk(z_hbm, out1d, zb_v, o_v,
                                             rl, 1))


@functools.lru_cache(maxsize=None)
def _extract_kernel(h):
    return pl.kernel(
        functools.partial(_extract_body, h=h),
        out_type=jax.ShapeDtypeStruct((EH,), jnp.float32),
        mesh=_mesh(),
        compiler_params=_sc_params(needs_layout_passes=False),
        scratch_types=[
            pltpu.VMEM((CHE * 8,), jnp.float32),
            pltpu.VMEM((CHE,), jnp.float32),
        ],
    )


def _edge_mlp_body(xr, xc, ea, g1r, g1c, g1e, b1, g2en, b2en,
                   gnr, gne, bn1, gm2, bm2, zout):
    zr = xr[...]
    zc = xc[...]
    ze = ea[...]
    dot = functools.partial(jnp.dot, preferred_element_type=jnp.float32)
    h = jnp.maximum(dot(zr, g1r[...]) + dot(zc, g1c[...])
                    + dot(ze, g1e[...]) + b1[...], 0.0)
    en = dot(h, g2en[...]) + b2en[...]
    hm = jnp.maximum(dot(zr, gnr[...]) + dot(en, gne[...]) + bn1[...], 0.0)
    zout[...] = en + dot(hm, gm2[...]) + bm2[...]


def _edge_mlp_last_body(xr, xc, ea, g1r, g1c, g1e, b1, g2en, b2en, zout):
    zr = xr[...]
    zc = xc[...]
    ze = ea[...]
    dot = functools.partial(jnp.dot, preferred_element_type=jnp.float32)
    h = jnp.maximum(dot(zr, g1r[...]) + dot(zc, g1c[...])
                    + dot(ze, g1e[...]) + b1[...], 0.0)
    zout[...] = dot(h, g2en[...]) + b2en[...]


_BR = 4000


def _edge_call(xr_p, xc_p, ea_p, w, last):
    espec = pl.BlockSpec((_BR, 128), lambda i: (i, 0))
    in_specs = [espec, espec, espec] + [
        pl.BlockSpec(a.shape, lambda i: (0, 0)) for a in w
    ]
    body = _edge_mlp_last_body if last else _edge_mlp_body
    return pl.pallas_call(
        body,
        grid=(EQH // _BR,),
        in_specs=in_specs,
        out_specs=espec,
        out_shape=jax.ShapeDtypeStruct((EQH, 128), jnp.float32),
    )(xr_p, xc_p, ea_p, *w)


def _node_body(xp, sa0, sa1, sb0, sb1, ca0, ca1, cb0, cb1,
               gx, gm, b1, g2, b2, xout):
    dot = functools.partial(jnp.dot, preferred_element_type=jnp.float32)
    cnt = jnp.maximum(ca0[...] + ca1[...] + cb0[...] + cb1[...], 1.0)
    mean = (sa0[...] + sa1[...] + sb0[...] + sb1[...]) / cnt
    h = jnp.maximum(dot(xp[...], gx[...]) + dot(mean, gm[...]) + b1[...], 0.0)
    xout[...] = dot(h, g2[...]) + b2[...]


_BN = 3128


def _node_call(xp, sa, sb, ca, cb, w):
    nspec = pl.BlockSpec((_BN, 128), lambda i: (i, 0))
    return pl.pallas_call(
        _node_body,
        grid=(NQ // _BN,),
        in_specs=[nspec] * 9 + [
            pl.BlockSpec(a.shape, lambda i: (0, 0)) for a in w
        ],
        out_specs=nspec,
        out_shape=jax.ShapeDtypeStruct((NQ, 128), jnp.float32),
    )(xp, sa[0], sa[1], sb[0], sb[1], ca[0], ca[1], cb[0], cb[1], *w)


def _place(w, r0, c0=0, rows=8, cols=16):
    out = jnp.zeros((rows, cols), jnp.float32)
    return out.at[r0:r0 + w.shape[0], c0:c0 + w.shape[1]].set(w)


def _blk(w):
    return jnp.kron(jnp.eye(16, dtype=jnp.float32), w)


def _edge_weights(lp, dx, last):
    w1 = lp["edge"]["W1"]
    g1r = _blk(_place(w1[0:dx], 0))
    g1c = _blk(_place(w1[dx:2 * dx], 0))
    g1e = _blk(_place(w1[2 * dx:], 0))
    b1 = jnp.tile(lp["edge"]["b1"], 16)[None]
    w2 = lp["edge"]["W2"]
    g2en = _blk(_place(w2, 0, 0, rows=16, cols=8))
    b2en = jnp.tile(_place(lp["edge"]["b2"][None], 0, 0, rows=1, cols=8)[0],
                    16)[None]
    ws = [g1r, g1c, g1e, b1, g2en, b2en]
    if last:
        return ws
    v1 = lp["node1"]["W1"]
    gnr = _blk(_place(v1[0:dx], 0))
    gne = _blk(_place(v1[dx:], 0))
    bn1 = jnp.tile(lp["node1"]["b1"], 16)[None]
    v2 = lp["node1"]["W2"]
    gm2 = _blk(_place(v2, 0, 4, rows=16, cols=8))
    bm2 = jnp.tile(_place(lp["node1"]["b2"][None], 0, 4, rows=1, cols=8)[0],
                   16)[None]
    return ws + [gnr, gne, bn1, gm2, bm2]


def _node_weights(lp, dx):
    u1 = lp["node2"]["W1"]
    gx = _blk(_place(u1[0:dx], 0))
    gm = _blk(_place(u1[dx:], 4))
    b1 = jnp.tile(lp["node2"]["b1"], 16)[None]
    u2 = lp["node2"]["W2"]
    g2 = _blk(_place(u2, 0, 0, rows=16, cols=8))
    b2 = jnp.tile(_place(lp["node2"]["b2"][None], 0, 0, rows=1, cols=8)[0],
                  16)[None]
    return [gx, gm, b1, g2, b2]


def kernel(x, edge_index, edge_attr, params):
    row2d = edge_index[0].reshape(ER, 128)
    col2d = edge_index[1].reshape(ER, 128)
    dx0 = x.shape[1]
    x_tbl = jnp.zeros((N_PAD, 8), jnp.float32).at[:N_NODES, :dx0].set(x)
    dea0 = edge_attr.shape[1]
    eaT = jnp.pad(edge_attr.T, ((0, 8 - dea0), (0, 0)))
    zeros1d = jnp.zeros((CHE * 8,), jnp.float32)
    ea_h = [_pack_kernel(h)(eaT, zeros1d).reshape(EQH, 128) for h in (0, 1)]
    zeros = jnp.zeros((N_PAD, 8), jnp.float32)
    ones = jnp.ones((128, 8), jnp.float32)

    cnt_h = None
    layers = params["layers"]
    dx = dx0
    for li, lp in enumerate(layers):
        last = li == len(layers) - 1
        ew = _edge_weights(lp, dx, last)
        z_h = [None, None]
        s_h = [None, None]
        c_h = [None, None] if cnt_h is None else cnt_h
        gathered = [_gather_kernel(h)(x_tbl, row2d, col2d) for h in (0, 1)]
        for h in (0, 1):
            xr, xc = gathered[h]
            z_h[h] = _edge_call(xr.reshape(EQH, 128), xc.reshape(EQH, 128),
                                ea_h[h], ew, last)
            if not last:
                if cnt_h is None:
                    s_h[h], c_h[h] = _scatter_kernel(h, True)(
                        z_h[h].reshape(EH, 8), col2d, zeros, ones)
                else:
                    res = _scatter_kernel(h, False)(
                        z_h[h].reshape(EH, 8), col2d, zeros)
                    s_h[h] = res[0] if isinstance(res, (list, tuple)) else res
        ea_h = z_h
        if last:
            break
        cnt_h = c_h
        nw = _node_weights(lp, dx)
        xq_new = _node_call(x_tbl.reshape(NQ, 128),
                            s_h[0].reshape(2, NQ, 128),
                            s_h[1].reshape(2, NQ, 128),
                            c_h[0].reshape(2, NQ, 128),
                            c_h[1].reshape(2, NQ, 128), nw)
        x_tbl = xq_new.reshape(N_PAD, 8)
        dx = 4

    outs = [_extract_kernel(h)(ea_h[h].reshape(EH * 8)) for h in (0, 1)]
    return jnp.concatenate(outs).reshape(E, 1)


# speedup vs baseline: 44.5305x; 1.0003x over previous
import functools

import jax
import jax.numpy as jnp
from jax import lax
from jax.experimental import pallas as pl
from jax.experimental.pallas import tpu as pltpu
from jax.experimental.pallas import tpu_sc as plsc

N_NODES = 100000
N_PAD = 100096
NQ = N_PAD // 16
E = 6400000
EH = E // 2
EQH = EH // 16
ER = E // 128
ERH = ER // 2
NW = 32
RPW = ERH // NW
KC = 11
NCH = RPW // KC
CHE = KC * 128
TAILN = ERH - NW * RPW
NSEG = N_PAD // 16


@functools.lru_cache(maxsize=None)
def _mesh():
    return plsc.VectorSubcoreMesh(core_axis_name="c", subcore_axis_name="s")


def _wid():
    return lax.axis_index("s") * 2 + lax.axis_index("c")


def _sc_params(**kw):
    return pltpu.CompilerParams(use_tc_tiling_on_sc=False, **kw)


def _half_loop(h, fn, tail_fn):
    w = _wid()
    base_loc = w * RPW

    def step(i, carry):
        r_loc = base_loc + i * KC
        fn(h * ERH + r_loc, r_loc)
        return carry

    lax.fori_loop(0, NCH, step, 0)

    @pl.when(w < TAILN)
    def _tail():
        r_loc = NW * RPW + w
        tail_fn(h * ERH + r_loc, r_loc)


def _gather_chunk(tbl, idx2d, out_hbm, idx_v, rows_v, sem,
                  r_abs, r_loc, nrows):
    pltpu.sync_copy(idx2d.at[pl.ds(r_abs, nrows), :],
                    idx_v.at[pl.ds(0, nrows), :])
    cps = []
    for j in range(nrows):
        cps.append(pltpu.async_copy(
            tbl.at[idx_v.at[j]],
            rows_v.at[pl.ds(j * 128, 128), :], sem))
    for cp in cps:
        cp.wait()
    pltpu.sync_copy(rows_v.at[pl.ds(0, nrows * 128), :],
                    out_hbm.at[pl.ds(r_loc * 128, nrows * 128), :])


def _gather_body(x_tbl, row2d, col2d, xr_out, xc_out,
                 tbl_s, idx_v, rows_v, idx2_v, rows2_v, sem1, sem2, h):
    s = lax.axis_index("s")
    pltpu.sync_copy(x_tbl.at[pl.ds(s * NSEG, NSEG), :],
                    tbl_s.at[pl.ds(s * NSEG, NSEG), :])
    plsc.subcore_barrier()

    def chunk(r_abs, r_loc, nrows):
        _gather_chunk(tbl_s, row2d, xr_out, idx_v, rows_v, sem1,
                      r_abs, r_loc, nrows)
        _gather_chunk(tbl_s, col2d, xc_out, idx2_v, rows2_v, sem2,
                      r_abs, r_loc, nrows)

    _half_loop(h,
               lambda ra, rl: chunk(ra, rl, KC),
               lambda ra, rl: chunk(ra, rl, 1))


@functools.lru_cache(maxsize=None)
def _gather_kernel(h):
    return pl.kernel(
        functools.partial(_gather_body, h=h),
        out_type=[jax.ShapeDtypeStruct((EH, 8), jnp.float32),
                  jax.ShapeDtypeStruct((EH, 8), jnp.float32)],
        mesh=_mesh(),
        compiler_params=_sc_params(),
        cost_estimate=pl.CostEstimate(flops=0, transcendentals=0,
                                      bytes_accessed=430_000_000),
        scratch_types=[
            pltpu.VMEM_SHARED((N_PAD, 8), jnp.float32),
            pltpu.VMEM((KC, 128), jnp.int32),
            pltpu.VMEM((CHE, 8), jnp.float32),
            pltpu.VMEM((KC, 128), jnp.int32),
            pltpu.VMEM((CHE, 8), jnp.float32),
            pltpu.SemaphoreType.DMA,
            pltpu.SemaphoreType.DMA,
        ],
    )


def _scatter_chunk(m_hbm, idx2d, acc_s, cnt_s, idx_v, upd_v, ones_v,
                   r_abs, r_loc, nrows, with_counts):
    pltpu.sync_copy(idx2d.at[pl.ds(r_abs, nrows), :],
                    idx_v.at[pl.ds(0, nrows), :])
    pltpu.sync_copy(m_hbm.at[pl.ds(r_loc * 128, nrows * 128), :],
                    upd_v.at[pl.ds(0, nrows * 128), :])
    for j in range(nrows):
        pltpu.sync_copy(upd_v.at[pl.ds(j * 128, 128), :],
                        acc_s.at[idx_v.at[j]], add=True)
        if with_counts:
            pltpu.sync_copy(ones_v, cnt_s.at[idx_v.at[j]], add=True)


def _scatter_body(m_hbm, col2d, zeros_hbm, ones_hbm, out_hbm, cnt_hbm,
                  acc_s, cnt_s, idx_v, upd_v, ones_v, h, with_counts):
    c = lax.axis_index("c")
    s = lax.axis_index("s")
    pltpu.sync_copy(zeros_hbm.at[pl.ds(s * NSEG, NSEG), :],
                    acc_s.at[pl.ds(s * NSEG, NSEG), :])
    if with_counts:
        pltpu.sync_copy(zeros_hbm.at[pl.ds(s * NSEG, NSEG), :],
                        cnt_s.at[pl.ds(s * NSEG, NSEG), :])
        pltpu.sync_copy(ones_hbm, ones_v)
    plsc.subcore_barrier()

    def chunk(r_abs, r_loc, nrows):
        _scatter_chunk(m_hbm, col2d, acc_s, cnt_s, idx_v, upd_v, ones_v,
                       r_abs, r_loc, nrows, with_counts)

    _half_loop(h,
               lambda ra, rl: chunk(ra, rl, KC),
               lambda ra, rl: chunk(ra, rl, 1))
    plsc.subcore_barrier()
    pltpu.sync_copy(acc_s.at[pl.ds(s * NSEG, NSEG), :],
                    out_hbm.at[c, pl.ds(s * NSEG, NSEG), :])
    if with_counts:
        pltpu.sync_copy(cnt_s.at[pl.ds(s * NSEG, NSEG), :],
                        cnt_hbm.at[c, pl.ds(s * NSEG, NSEG), :])


@functools.lru_cache(maxsize=None)
def _scatter_kernel(h, with_counts):
    n_out = 2 if with_counts else 1
    if with_counts:
        def body(m_hbm, col2d, zeros_hbm, ones_hbm, out_hbm, cnt_hbm,
                 acc_s, cnt_s, idx_v, upd_v, ones_v):
            return _scatter_body(m_hbm, col2d, zeros_hbm, ones_hbm,
                                 out_hbm, cnt_hbm,
                                 acc_s, cnt_s, idx_v, upd_v, ones_v,
                                 h, True)
    else:
        def body(m_hbm, col2d, zeros_hbm, out_hbm,
                 acc_s, idx_v, upd_v):
            return _scatter_body(m_hbm, col2d, zeros_hbm, None,
                                 out_hbm, None,
                                 acc_s, None, idx_v, upd_v, None,
                                 h, False)
    out_type = [jax.ShapeDtypeStruct((2, N_PAD, 8), jnp.float32)] * n_out
    scratch = [pltpu.VMEM_SHARED((N_PAD, 8), jnp.float32)]
    if with_counts:
        scratch.append(pltpu.VMEM_SHARED((N_PAD, 8), jnp.float32))
    scratch += [
        pltpu.VMEM((KC, 128), jnp.int32),
        pltpu.VMEM((CHE, 8), jnp.float32),
    ]
    if with_counts:
        scratch.append(pltpu.VMEM((128, 8), jnp.float32))
    return pl.kernel(body, out_type=out_type, mesh=_mesh(),
                     compiler_params=_sc_params(),
                     cost_estimate=pl.CostEstimate(flops=0, transcendentals=0,
                                                   bytes_accessed=150_000_000),
                     scratch_types=scratch)


def _pack_chunk(eaT, z0_out, pln_v, zb_v, r_abs, r_loc, nrows):
    n = nrows * 128
    for f in range(3):
        pltpu.sync_copy(eaT.at[f, pl.ds(r_abs * 128, n)],
                        pln_v.at[f, pl.ds(0, n)])
    lanes = lax.iota(jnp.int32, 16) * 8

    def grp(g, carry):
        base8 = g * 128
        for f in range(3):
            v = pln_v[f, pl.ds(g * 16, 16)]
            plsc.store_scatter(zb_v, [lanes + (base8 + f)], v)
        return carry

    lax.fori_loop(0, nrows * 8, grp, 0)
    pltpu.sync_copy(zb_v.at[pl.ds(0, n * 8)],
                    z0_out.at[pl.ds(r_loc * 1024, n * 8)])


def _pack_body(eaT, zeros1d, z0_out, pln_v, zb_v, h):
    pltpu.sync_copy(zeros1d, zb_v)
    _half_loop(h,
               lambda ra, rl: _pack_chunk(eaT, z0_out, pln_v, zb_v,
                                          ra, rl, KC),
               lambda ra, rl: _pack_chunk(eaT, z0_out, pln_v, zb_v,
                                          ra, rl, 1))


@functools.lru_cache(maxsize=None)
def _pack_kernel(h):
    return pl.kernel(
        functools.partial(_pack_body, h=h),
        out_type=jax.ShapeDtypeStruct((EH * 8,), jnp.float32),
        mesh=_mesh(),
        compiler_params=_sc_params(needs_layout_passes=False),
        scratch_types=[
            pltpu.VMEM((3, CHE), jnp.float32),
            pltpu.VMEM((CHE * 8,), jnp.float32),
        ],
    )


def _extract_chunk(z_hbm, out1d, zb_v, o_v, r_loc, nrows):
    n = nrows * 128
    pltpu.sync_copy(z_hbm.at[pl.ds(r_loc * 1024, n * 8)],
                    zb_v.at[pl.ds(0, n * 8)])
    lanes = lax.iota(jnp.int32, 16) * 8

    def grp(g, carry):
        v = plsc.load_gather(zb_v, [lanes + g * 128])
        o_v[pl.ds(g * 16, 16)] = v
        return carry

    lax.fori_loop(0, nrows * 8, grp, 0)
    pltpu.sync_copy(o_v.at[pl.ds(0, n)], out1d.at[pl.ds(r_loc * 128, n)])


def _extract_body(z_hbm, out1d, zb_v, o_v, h):
    _half_loop(h,
               lambda ra, rl: _extract_chunk(z_hbm, out1d, zb_v, o_v,
                                             rl, KC),
               lambda ra, rl: _extract_chunk(z_hbm, out1d, zb_v, o_v,
                                             rl, 1))


@functools.lru_cache(maxsize=None)
def _extract_kernel(h):
    return pl.kernel(
        functools.partial(_extract_body, h=h),
        out_type=jax.ShapeDtypeStruct((EH,), jnp.float32),
        mesh=_mesh(),
        compiler_params=_sc_params(needs_layout_passes=False),
        scratch_types=[
            pltpu.VMEM((CHE * 8,), jnp.float32),
            pltpu.VMEM((CHE,), jnp.float32),
        ],
    )


def _edge_mlp_body(xr, xc, ea, g1r, g1c, g1e, b1, g2en, b2en,
                   gnr, gne, bn1, gm2, bm2, zout):
    zr = xr[...]
    zc = xc[...]
    ze = ea[...]
    dot = functools.partial(jnp.dot, preferred_element_type=jnp.float32)
    h = jnp.maximum(dot(zr, g1r[...]) + dot(zc, g1c[...])
                    + dot(ze, g1e[...]) + b1[...], 0.0)
    en = dot(h, g2en[...]) + b2en[...]
    hm = jnp.maximum(dot(zr, gnr[...]) + dot(en, gne[...]) + bn1[...], 0.0)
    zout[...] = en + dot(hm, gm2[...]) + bm2[...]


def _edge_mlp_last_body(xr, xc, ea, g1r, g1c, g1e, b1, g2en, b2en, zout):
    zr = xr[...]
    zc = xc[...]
    ze = ea[...]
    dot = functools.partial(jnp.dot, preferred_element_type=jnp.float32)
    h = jnp.maximum(dot(zr, g1r[...]) + dot(zc, g1c[...])
                    + dot(ze, g1e[...]) + b1[...], 0.0)
    zout[...] = dot(h, g2en[...]) + b2en[...]


_BR = 4000


def _edge_call(xr_p, xc_p, ea_p, w, last):
    espec = pl.BlockSpec((_BR, 128), lambda i: (i, 0))
    in_specs = [espec, espec, espec] + [
        pl.BlockSpec(a.shape, lambda i: (0, 0)) for a in w
    ]
    body = _edge_mlp_last_body if last else _edge_mlp_body
    return pl.pallas_call(
        body,
        grid=(EQH // _BR,),
        in_specs=in_specs,
        out_specs=espec,
        out_shape=jax.ShapeDtypeStruct((EQH, 128), jnp.float32),
    )(xr_p, xc_p, ea_p, *w)


def _node_body(xp, sa0, sa1, sb0, sb1, ca0, ca1, cb0, cb1,
               gx, gm, b1, g2, b2, xout):
    dot = functools.partial(jnp.dot, preferred_element_type=jnp.float32)
    cnt = jnp.maximum(ca0[...] + ca1[...] + cb0[...] + cb1[...], 1.0)
    mean = (sa0[...] + sa1[...] + sb0[...] + sb1[...]) / cnt
    h = jnp.maximum(dot(xp[...], gx[...]) + dot(mean, gm[...]) + b1[...], 0.0)
    xout[...] = dot(h, g2[...]) + b2[...]


_BN = 3128


def _node_call(xp, sa, sb, ca, cb, w):
    nspec = pl.BlockSpec((_BN, 128), lambda i: (i, 0))
    return pl.pallas_call(
        _node_body,
        grid=(NQ // _BN,),
        in_specs=[nspec] * 9 + [
            pl.BlockSpec(a.shape, lambda i: (0, 0)) for a in w
        ],
        out_specs=nspec,
        out_shape=jax.ShapeDtypeStruct((NQ, 128), jnp.float32),
    )(xp, sa[0], sa[1], sb[0], sb[1], ca[0], ca[1], cb[0], cb[1], *w)


def _place(w, r0, c0=0, rows=8, cols=16):
    out = jnp.zeros((rows, cols), jnp.float32)
    return out.at[r0:r0 + w.shape[0], c0:c0 + w.shape[1]].set(w)


def _blk(w):
    return jnp.kron(jnp.eye(16, dtype=jnp.float32), w)


def _edge_weights(lp, dx, last):
    w1 = lp["edge"]["W1"]
    g1r = _blk(_place(w1[0:dx], 0))
    g1c = _blk(_place(w1[dx:2 * dx], 0))
    g1e = _blk(_place(w1[2 * dx:], 0))
    b1 = jnp.tile(lp["edge"]["b1"], 16)[None]
    w2 = lp["edge"]["W2"]
    g2en = _blk(_place(w2, 0, 0, rows=16, cols=8))
    b2en = jnp.tile(_place(lp["edge"]["b2"][None], 0, 0, rows=1, cols=8)[0],
                    16)[None]
    ws = [g1r, g1c, g1e, b1, g2en, b2en]
    if last:
        return ws
    v1 = lp["node1"]["W1"]
    gnr = _blk(_place(v1[0:dx], 0))
    gne = _blk(_place(v1[dx:], 0))
    bn1 = jnp.tile(lp["node1"]["b1"], 16)[None]
    v2 = lp["node1"]["W2"]
    gm2 = _blk(_place(v2, 0, 4, rows=16, cols=8))
    bm2 = jnp.tile(_place(lp["node1"]["b2"][None], 0, 4, rows=1, cols=8)[0],
                   16)[None]
    return ws + [gnr, gne, bn1, gm2, bm2]


def _node_weights(lp, dx):
    u1 = lp["node2"]["W1"]
    gx = _blk(_place(u1[0:dx], 0))
    gm = _blk(_place(u1[dx:], 4))
    b1 = jnp.tile(lp["node2"]["b1"], 16)[None]
    u2 = lp["node2"]["W2"]
    g2 = _blk(_place(u2, 0, 0, rows=16, cols=8))
    b2 = jnp.tile(_place(lp["node2"]["b2"][None], 0, 0, rows=1, cols=8)[0],
                  16)[None]
    return [gx, gm, b1, g2, b2]


def kernel(x, edge_index, edge_attr, params):
    row2d = edge_index[0].reshape(ER, 128)
    col2d = edge_index[1].reshape(ER, 128)
    dx0 = x.shape[1]
    x_tbl = jnp.zeros((N_PAD, 8), jnp.float32).at[:N_NODES, :dx0].set(x)
    dea0 = edge_attr.shape[1]
    eaT = jnp.pad(edge_attr.T, ((0, 8 - dea0), (0, 0)))
    zeros1d = jnp.zeros((CHE * 8,), jnp.float32)
    ea_h = [_pack_kernel(h)(eaT, zeros1d).reshape(EQH, 128) for h in (0, 1)]
    zeros = jnp.zeros((N_PAD, 8), jnp.float32)
    ones = jnp.ones((128, 8), jnp.float32)

    cnt_h = None
    layers = params["layers"]
    dx = dx0
    for li, lp in enumerate(layers):
        last = li == len(layers) - 1
        ew = _edge_weights(lp, dx, last)
        z_h = [None, None]
        s_h = [None, None]
        c_h = [None, None] if cnt_h is None else cnt_h
        gathered = [_gather_kernel(h)(x_tbl, row2d, col2d) for h in (0, 1)]
        for h in (0, 1):
            xr, xc = gathered[h]
            z_h[h] = _edge_call(xr.reshape(EQH, 128), xc.reshape(EQH, 128),
                                ea_h[h], ew, last)
            if not last:
                if cnt_h is None:
                    s_h[h], c_h[h] = _scatter_kernel(h, True)(
                        z_h[h].reshape(EH, 8), col2d, zeros, ones)
                else:
                    res = _scatter_kernel(h, False)(
                        z_h[h].reshape(EH, 8), col2d, zeros)
                    s_h[h] = res[0] if isinstance(res, (list, tuple)) else res
        ea_h = z_h
        if last:
            break
        cnt_h = c_h
        nw = _node_weights(lp, dx)
        xq_new = _node_call(x_tbl.reshape(NQ, 128),
                            s_h[0].reshape(2, NQ, 128),
                            s_h[1].reshape(2, NQ, 128),
                            c_h[0].reshape(2, NQ, 128),
                            c_h[1].reshape(2, NQ, 128), nw)
        x_tbl = xq_new.reshape(N_PAD, 8)
        dx = 4

    outs = [_extract_kernel(h)(ea_h[h].reshape(EH * 8)) for h in (0, 1)]
    return jnp.concatenate(outs).reshape(E, 1)
